# baseline (speedup 1.0000x reference)
import functools

import jax
import jax.numpy as jnp
from jax import lax
from jax.experimental import pallas as pl
from jax.experimental.pallas import tpu as pltpu
from jax.experimental.pallas import tpu_sc as plsc

NC = 2
NS = 16
NW = NC * NS
LANES = 16
CHUNK = 128


def _make_sc_kernel(n_rows, max_len, emb_dim):
    assert n_rows % (NW * CHUNK) == 0
    assert emb_dim % LANES == 0
    rows_per_w = n_rows // NW
    chunks_per_w = rows_per_w // CHUNK
    n_slices = emb_dim // LANES
    mesh = plsc.VectorSubcoreMesh(core_axis_name="c", subcore_axis_name="s")

    @functools.partial(
        pl.kernel,
        mesh=mesh,
        out_type=jax.ShapeDtypeStruct((n_rows, emb_dim), jnp.float32),
        scratch_types=[
            pltpu.VMEM((chunks_per_w, CHUNK), jnp.int32),
            pltpu.VMEM((CHUNK, emb_dim), jnp.float32),
            pltpu.VMEM((2 * max_len, emb_dim), jnp.float32),
            pltpu.SemaphoreType.DMA,
        ],
        compiler_params=pltpu.CompilerParams(use_tc_tiling_on_sc=False),
    )
    def k(x_hbm, table_hbm, pe_hbm, out_hbm, idx_v, rows_v, pe_v, sem):
        wid = lax.axis_index("s") * NC + lax.axis_index("c")
        base = wid * rows_per_w
        pltpu.sync_copy(x_hbm.at[pl.ds(wid * chunks_per_w, chunks_per_w)], idx_v)
        pltpu.sync_copy(pe_hbm, pe_v.at[pl.ds(0, max_len)])
        pltpu.sync_copy(pe_hbm, pe_v.at[pl.ds(max_len, max_len)])

        def chunk_body(c, carry):
            pltpu.async_copy(table_hbm.at[idx_v.at[c]], rows_v, sem).wait()
            start = lax.rem(c * CHUNK, max_len)

            def add_row(r, carry2):
                p = start + r
                for j in range(n_slices):
                    sl = pl.ds(j * LANES, LANES)
                    rows_v[r, sl] = rows_v[r, sl] + pe_v[p, sl]
                return carry2

            lax.fori_loop(0, CHUNK, add_row, 0, unroll=2)
            pltpu.sync_copy(rows_v, out_hbm.at[pl.ds(base + c * CHUNK, CHUNK)])
            return carry

        lax.fori_loop(0, chunks_per_w, chunk_body, 0)

    return k


def kernel(x, table, pe):
    batch, max_len = x.shape
    n_vocab, emb_dim = table.shape
    n_rows = batch * max_len
    x_flat = x.reshape(n_rows // CHUNK, CHUNK).astype(jnp.int32)
    pe2d = pe.reshape(max_len, emb_dim).astype(jnp.float32)
    k = _make_sc_kernel(n_rows, max_len, emb_dim)
    out = k(x_flat, table, pe2d)
    return out.reshape(batch, max_len, emb_dim)


# speedup vs baseline: 1.1289x; 1.1289x over previous
import functools

import jax
import jax.numpy as jnp
from jax import lax
from jax.experimental import pallas as pl
from jax.experimental.pallas import tpu as pltpu
from jax.experimental.pallas import tpu_sc as plsc

NC = 2
NS = 16
NW = NC * NS
LANES = 16
CHUNK = 128


def _make_sc_kernel(n_rows, max_len, emb_dim):
    assert n_rows % (NW * CHUNK) == 0
    assert emb_dim % LANES == 0
    rows_per_w = n_rows // NW
    chunks_per_w = rows_per_w // CHUNK
    assert chunks_per_w % 2 == 0
    n_slices = emb_dim // LANES
    mesh = plsc.VectorSubcoreMesh(core_axis_name="c", subcore_axis_name="s")

    @functools.partial(
        pl.kernel,
        mesh=mesh,
        out_type=jax.ShapeDtypeStruct((n_rows, emb_dim), jnp.float32),
        scratch_types=[
            pltpu.VMEM((chunks_per_w, CHUNK), jnp.int32),
            pltpu.VMEM((CHUNK, emb_dim), jnp.float32),
            pltpu.VMEM((CHUNK, emb_dim), jnp.float32),
            pltpu.VMEM((2 * max_len, emb_dim), jnp.float32),
            pltpu.SemaphoreType.DMA,
            pltpu.SemaphoreType.DMA,
        ],
        compiler_params=pltpu.CompilerParams(use_tc_tiling_on_sc=False),
    )
    def k(x_hbm, table_hbm, pe_hbm, out_hbm, idx_v, buf0, buf1, pe_v, sem0, sem1):
        wid = lax.axis_index("s") * NC + lax.axis_index("c")
        base = wid * rows_per_w
        pltpu.sync_copy(x_hbm.at[pl.ds(wid * chunks_per_w, chunks_per_w)], idx_v)
        pltpu.sync_copy(pe_hbm, pe_v.at[pl.ds(0, max_len)])
        pltpu.sync_copy(pe_hbm, pe_v.at[pl.ds(max_len, max_len)])
        bufs = (buf0, buf1)
        sems = (sem0, sem1)

        def start_gather(c, b):
            pltpu.async_copy(table_hbm.at[idx_v.at[c]], bufs[b], sems[b])

        def finish_chunk(c, b):
            buf = bufs[b]
            pltpu.make_async_copy(table_hbm.at[idx_v.at[c]], buf, sems[b]).wait()
            start = lax.rem(c * CHUNK, max_len)

            def add_row(r, carry):
                p = start + r
                for j in range(n_slices):
                    sl = pl.ds(j * LANES, LANES)
                    buf[r, sl] = buf[r, sl] + pe_v[p, sl]
                return carry

            lax.fori_loop(0, CHUNK, add_row, 0, unroll=2)
            pltpu.sync_copy(buf, out_hbm.at[pl.ds(base + c * CHUNK, CHUNK)])

        start_gather(0, 0)

        def pair_body(p, carry):
            c = p * 2
            start_gather(c + 1, 1)
            finish_chunk(c, 0)

            @pl.when(c + 2 < chunks_per_w)
            def _():
                start_gather(c + 2, 0)

            finish_chunk(c + 1, 1)
            return carry

        lax.fori_loop(0, chunks_per_w // 2, pair_body, 0)

    return k


def kernel(x, table, pe):
    batch, max_len = x.shape
    n_vocab, emb_dim = table.shape
    n_rows = batch * max_len
    x_flat = x.reshape(n_rows // CHUNK, CHUNK).astype(jnp.int32)
    pe2d = pe.reshape(max_len, emb_dim).astype(jnp.float32)
    k = _make_sc_kernel(n_rows, max_len, emb_dim)
    out = k(x_flat, table, pe2d)
    return out.reshape(batch, max_len, emb_dim)


# speedup vs baseline: 1.4919x; 1.3216x over previous
import functools

import jax
import jax.numpy as jnp
from jax import lax
from jax.experimental import pallas as pl
from jax.experimental.pallas import tpu as pltpu
from jax.experimental.pallas import tpu_sc as plsc

NC = 2
NS = 16
NW = NC * NS
LANES = 16
CHUNK = 128
NG = 4
NO = 2


def _make_sc_kernel(n_rows, max_len, emb_dim):
    assert n_rows % (NW * CHUNK) == 0
    assert emb_dim % LANES == 0
    rows_per_w = n_rows // NW
    chunks_per_w = rows_per_w // CHUNK
    assert chunks_per_w % NG == 0
    n_slices = emb_dim // LANES
    mesh = plsc.VectorSubcoreMesh(core_axis_name="c", subcore_axis_name="s")

    @functools.partial(
        pl.kernel,
        mesh=mesh,
        out_type=jax.ShapeDtypeStruct((n_rows, emb_dim), jnp.float32),
        scratch_types=[
            pltpu.VMEM((chunks_per_w, CHUNK), jnp.int32),
            [pltpu.VMEM((CHUNK, emb_dim), jnp.float32) for _ in range(NG)],
            [pltpu.VMEM((CHUNK, emb_dim), jnp.float32) for _ in range(NO)],
            pltpu.VMEM((2 * max_len, emb_dim), jnp.float32),
            [pltpu.SemaphoreType.DMA for _ in range(NG)],
            [pltpu.SemaphoreType.DMA for _ in range(NO)],
        ],
        compiler_params=pltpu.CompilerParams(use_tc_tiling_on_sc=False),
    )
    def k(x_hbm, table_hbm, pe_hbm, out_hbm, idx_v, gbufs, obufs, pe_v,
          gsems, ssems):
        wid = lax.axis_index("s") * NC + lax.axis_index("c")
        base = wid * rows_per_w
        pltpu.sync_copy(x_hbm.at[pl.ds(wid * chunks_per_w, chunks_per_w)], idx_v)
        pltpu.sync_copy(pe_hbm, pe_v.at[pl.ds(0, max_len)])
        pltpu.sync_copy(pe_hbm, pe_v.at[pl.ds(max_len, max_len)])

        def start_gather(c, g):
            pltpu.async_copy(table_hbm.at[idx_v.at[c]], gbufs[g], gsems[g])

        def out_slice(c):
            return out_hbm.at[pl.ds(base + c * CHUNK, CHUNK)]

        for g in range(NG - 1):
            start_gather(g, g)

        def chunk_step(c, g, o):
            @pl.when(c + NG - 1 < chunks_per_w)
            def _():
                start_gather(c + NG - 1, (g + NG - 1) % NG)

            gbuf, obuf = gbufs[g], obufs[o]
            pltpu.make_async_copy(table_hbm.at[idx_v.at[c]], gbuf, gsems[g]).wait()

            @pl.when(c >= NO)
            def _():
                pltpu.make_async_copy(obuf, out_slice(c - NO), ssems[o]).wait()

            start = lax.rem(c * CHUNK, max_len)

            @plsc.parallel_loop(0, CHUNK, step=1, unroll=4)
            def _(r):
                p = start + r
                for j in range(n_slices):
                    sl = pl.ds(j * LANES, LANES)
                    obuf[r, sl] = gbuf[r, sl] + pe_v[p, sl]
            pltpu.async_copy(obuf, out_slice(c), ssems[o])

        def group_body(q, carry):
            c0 = q * NG
            for b in range(NG):
                chunk_step(c0 + b, b, b % NO)
            return carry

        lax.fori_loop(0, chunks_per_w // NG, group_body, 0)

        for b in range(NO):
            c = chunks_per_w - NO + b
            pltpu.make_async_copy(obufs[c % NO], out_slice(c), ssems[c % NO]).wait()

    return k


def kernel(x, table, pe):
    batch, max_len = x.shape
    n_vocab, emb_dim = table.shape
    n_rows = batch * max_len
    x_flat = x.reshape(n_rows // CHUNK, CHUNK).astype(jnp.int32)
    pe2d = pe.reshape(max_len, emb_dim).astype(jnp.float32)
    k = _make_sc_kernel(n_rows, max_len, emb_dim)
    out = k(x_flat, table, pe2d)
    return out.reshape(batch, max_len, emb_dim)
